# Initial kernel scaffold; baseline (speedup 1.0000x reference)
import functools

import jax
import jax.numpy as jnp
from jax import lax
from jax.experimental import pallas as pl
from jax.experimental.pallas import tpu as pltpu
from jax.experimental.pallas import tpu_sc as plsc

B = 4
T1 = 1024
C = 1024
RM = 512
TOUT = 2 * T1 - RM
TRASH = B * TOUT
OUT_ROWS = TRASH + 8

NS = 16
RPT = T1 // NS
CH = 16


def _prep_body(a_ref, b_ref, as_ref, bs_ref, p_ref, d_ref):
    bi = pl.program_id(0)
    a = a_ref[0]
    bb = b_ref[0]
    scores = lax.dot_general(a, bb, (((1,), (1,)), ((), ())),
                             preferred_element_type=jnp.float32)
    s_t = lax.transpose(scores, (1, 0))
    lane = lax.broadcasted_iota(jnp.int32, (T1, T1), 1)
    sub = lax.broadcasted_iota(jnp.int32, (T1, T1), 0)
    big = jnp.int32(T1)

    kcol = jnp.max(scores, axis=1, keepdims=True)
    icol = jnp.min(jnp.where(scores == kcol, lane, big), axis=1, keepdims=True)
    krow = jnp.max(s_t, axis=0, keepdims=True)
    irow = jnp.min(jnp.where(s_t == krow, sub, big), axis=0, keepdims=True)

    mc = (krow > kcol) | ((krow == kcol) & (lane < sub))
    rcol = jnp.sum(mc.astype(jnp.int32), axis=1, keepdims=True)
    mr = (kcol > krow) | ((kcol == krow) & (sub < lane))
    rrow = jnp.sum(mr.astype(jnp.int32), axis=0, keepdims=True)

    mgcol = rcol < RM
    mgrow = rrow < RM

    ccol = 1.0 + jnp.sum(((irow == sub) & mgrow).astype(jnp.float32),
                         axis=1, keepdims=True)
    crow = 1.0 + jnp.sum(((icol == lane) & mgcol).astype(jnp.float32),
                         axis=0, keepdims=True)
    invcol = 1.0 / ccol
    invrow = 1.0 / crow

    sraw = jnp.sum(jnp.where(icol == lane, invrow, 0.0), axis=1, keepdims=True)
    scl = jnp.where(mgcol, sraw, 1.0)

    as_ref[0] = a * scl
    bs_ref[0] = bb * invcol
    p_ref[0] = jnp.where(mgrow, jnp.int32(TRASH), bi * TOUT + rrow - RM)
    d_ref[0] = jnp.where(mgrow, irow, jnp.int32(T1))


def _prep(a, b):
    return pl.pallas_call(
        _prep_body,
        grid=(B,),
        in_specs=[pl.BlockSpec((1, T1, C), lambda i: (i, 0, 0)),
                  pl.BlockSpec((1, T1, C), lambda i: (i, 0, 0))],
        out_specs=[pl.BlockSpec((1, T1, C), lambda i: (i, 0, 0)),
                   pl.BlockSpec((1, T1, C), lambda i: (i, 0, 0)),
                   pl.BlockSpec((1, 1, T1), lambda i: (i, 0, 0)),
                   pl.BlockSpec((1, 1, T1), lambda i: (i, 0, 0))],
        out_shape=[jax.ShapeDtypeStruct((B, T1, C), jnp.float32),
                   jax.ShapeDtypeStruct((B, T1, C), jnp.float32),
                   jax.ShapeDtypeStruct((B, 1, T1), jnp.int32),
                   jax.ShapeDtypeStruct((B, 1, T1), jnp.int32)],
    )(a, b)


@functools.partial(
    pl.kernel,
    mesh=plsc.VectorSubcoreMesh(core_axis_name="c", subcore_axis_name="s"),
    out_type=jax.ShapeDtypeStruct((OUT_ROWS, C), jnp.float32),
    scratch_types=[pltpu.VMEM((CH, C), jnp.float32),
                   pltpu.VMEM((2, RPT), jnp.int32),
                   pltpu.VMEM_SHARED((T1 + 8, C), jnp.float32)],
)
def _merge(as_hbm, bs_hbm, p_hbm, d_hbm, out_hbm, buf, idxbuf, accum):
    c = lax.axis_index("c")
    s = lax.axis_index("s")
    row0 = s * RPT
    for t in range(2):
        batch = c * 2 + t
        base = batch * T1
        for k in range(0, RPT, CH):
            pltpu.sync_copy(bs_hbm.at[pl.ds(base + row0 + k, CH)], buf)
            pltpu.sync_copy(buf, accum.at[pl.ds(row0 + k, CH)])
        pltpu.sync_copy(p_hbm.at[pl.ds(base + row0, RPT)], idxbuf.at[0])
        pltpu.sync_copy(d_hbm.at[pl.ds(base + row0, RPT)], idxbuf.at[1])
        plsc.subcore_barrier()
        for k in range(0, RPT, CH):
            pltpu.sync_copy(as_hbm.at[pl.ds(base + row0 + k, CH)], buf)
            pvec = idxbuf[0, pl.ds(k, CH)]
            dvec = idxbuf[1, pl.ds(k, CH)]
            pltpu.sync_copy(buf, out_hbm.at[pvec])
            pltpu.sync_copy(buf, accum.at[dvec], add=True)
        plsc.subcore_barrier()
        obase = batch * TOUT + RM
        for k in range(0, RPT, CH):
            pltpu.sync_copy(accum.at[pl.ds(row0 + k, CH)], buf)
            pltpu.sync_copy(buf, out_hbm.at[pl.ds(obase + row0 + k, CH)])
        plsc.subcore_barrier()


def kernel(x):
    a = x[:, ::2, :]
    b = x[:, 1::2, :]
    a_s, b_s, p, d = _prep(a, b)
    out = _merge(a_s.reshape(B * T1, C), b_s.reshape(B * T1, C),
                 p.reshape(B * T1), d.reshape(B * T1))
    return out[:TRASH].reshape(B, TOUT, C)


# speedup vs baseline: 2.6531x; 2.6531x over previous
import functools

import jax
import jax.numpy as jnp
from jax import lax
from jax.experimental import pallas as pl
from jax.experimental.pallas import tpu as pltpu
from jax.experimental.pallas import tpu_sc as plsc

B = 4
T1 = 1024
C = 1024
RM = 512
TOUT = 2 * T1 - RM
UNM = T1 - RM

NS = 16
NW = 2 * NS
RPW = B * UNM // NW


def _prep_body(a_ref, b_ref, bn_ref, src_ref):
    bi = pl.program_id(0)
    a = a_ref[0]
    bb = b_ref[0]
    scores = lax.dot_general(a, bb, (((1,), (1,)), ((), ())),
                             preferred_element_type=jnp.float32)
    s_t = lax.transpose(scores, (1, 0))
    lane = lax.broadcasted_iota(jnp.int32, (T1, T1), 1)
    sub = lax.broadcasted_iota(jnp.int32, (T1, T1), 0)
    big = jnp.int32(T1)

    kcol = jnp.max(scores, axis=1, keepdims=True)
    krow = jnp.max(s_t, axis=0, keepdims=True)
    irow = jnp.min(jnp.where(s_t == krow, sub, big), axis=0, keepdims=True)

    mc = (krow > kcol) | ((krow == kcol) & (lane < sub))
    rcol = jnp.sum(mc.astype(jnp.int32), axis=1, keepdims=True)
    mr = (kcol > krow) | ((kcol == krow) & (sub < lane))
    rrow = jnp.sum(mr.astype(jnp.int32), axis=0, keepdims=True)
    mgrow = rrow < RM

    w = ((irow == sub) & mgrow).astype(jnp.float32)
    msum = lax.dot_general(w, a, (((1,), (0,)), ((), ())),
                           preferred_element_type=jnp.float32)
    counts = 1.0 + jnp.sum(w, axis=1, keepdims=True)
    bn_ref[0] = (bb + msum) / counts

    inv = jnp.sum(jnp.where(rcol == lane, sub, 0), axis=0, keepdims=True)
    src_ref[0] = bi * (2 * T1) + 2 * inv


def _prep(a, b):
    return pl.pallas_call(
        _prep_body,
        grid=(B,),
        in_specs=[pl.BlockSpec((1, T1, C), lambda i: (i, 0, 0)),
                  pl.BlockSpec((1, T1, C), lambda i: (i, 0, 0))],
        out_specs=[pl.BlockSpec((1, T1, C), lambda i: (i, 0, 0)),
                   pl.BlockSpec((1, 1, T1), lambda i: (i, 0, 0))],
        out_shape=[jax.ShapeDtypeStruct((B, T1, C), jnp.float32),
                   jax.ShapeDtypeStruct((B, 1, T1), jnp.int32)],
    )(a, b)


def _gather_body(x_hbm, src_hbm, out_hbm, buf, idx, sem):
    c = lax.axis_index("c")
    s = lax.axis_index("s")
    wid = s * 2 + c
    base = wid * RPW
    pltpu.sync_copy(src_hbm.at[pl.ds(base, RPW)], idx)
    pltpu.async_copy(x_hbm.at[idx], buf, sem).wait()
    pltpu.sync_copy(buf, out_hbm.at[pl.ds(base, RPW)])


@functools.cache
def _make_gather():
    return functools.partial(
        pl.kernel,
        mesh=plsc.VectorSubcoreMesh(core_axis_name="c", subcore_axis_name="s"),
        out_type=jax.ShapeDtypeStruct((B * UNM, C), jnp.float32),
        scratch_types=[pltpu.VMEM((RPW, C), jnp.float32),
                       pltpu.VMEM((RPW,), jnp.int32),
                       pltpu.SemaphoreType.DMA],
    )(_gather_body)


def kernel(x):
    a = x[:, ::2, :]
    b = x[:, 1::2, :]
    b_new, src = _prep(a, b)
    src_unm = src.reshape(B, T1)[:, RM:].reshape(B * UNM)
    unm = _make_gather()(x.reshape(2 * B * T1, C), src_unm)
    return jnp.concatenate([unm.reshape(B, UNM, C), b_new], axis=1)


# speedup vs baseline: 4.9313x; 1.8587x over previous
import functools

import jax
import jax.numpy as jnp
from jax import lax
from jax.experimental import pallas as pl
from jax.experimental.pallas import tpu as pltpu
from jax.experimental.pallas import tpu_sc as plsc

B = 4
T1 = 1024
C = 1024
RM = 512
TOUT = 2 * T1 - RM
UNM = T1 - RM

NS = 16
NW = 2 * NS
RPW = B * UNM // NW


def _prep_body(x_ref, bn_ref, src_ref):
    bi = pl.program_id(0)
    a = x_ref[0, :, 0, :]
    bb = x_ref[0, :, 1, :]
    scores = lax.dot_general(a, bb, (((1,), (1,)), ((), ())),
                             preferred_element_type=jnp.float32)
    lane = lax.broadcasted_iota(jnp.int32, (T1, T1), 1)
    sub = lax.broadcasted_iota(jnp.int32, (T1, T1), 0)
    big = jnp.int32(T1)

    kcol = jnp.max(scores, axis=1, keepdims=True)
    krow = lax.transpose(kcol, (1, 0))
    icol = jnp.min(jnp.where(scores == kcol, lane, big), axis=1, keepdims=True)
    irow = lax.transpose(icol, (1, 0))

    mc = (krow > kcol) | ((krow == kcol) & (lane < sub))
    rcol = jnp.sum(mc.astype(jnp.int32), axis=1, keepdims=True)
    mgrow = lax.transpose(rcol, (1, 0)) < RM

    w = ((irow == sub) & mgrow).astype(jnp.float32)
    msum = lax.dot_general(w, a, (((1,), (0,)), ((), ())),
                           preferred_element_type=jnp.float32)
    counts = 1.0 + jnp.sum(w, axis=1, keepdims=True)
    bn_ref[0] = (bb + msum) / counts

    inv = jnp.sum(jnp.where(rcol == lane, sub, 0), axis=0, keepdims=True)
    src_ref[0] = bi * (2 * T1) + 2 * inv


def _prep(x4):
    return pl.pallas_call(
        _prep_body,
        grid=(B,),
        in_specs=[pl.BlockSpec((1, T1, 2, C), lambda i: (i, 0, 0, 0))],
        out_specs=[pl.BlockSpec((1, T1, C), lambda i: (i, 0, 0)),
                   pl.BlockSpec((1, 1, T1), lambda i: (i, 0, 0))],
        out_shape=[jax.ShapeDtypeStruct((B, T1, C), jnp.float32),
                   jax.ShapeDtypeStruct((B, 1, T1), jnp.int32)],
    )(x4)


def _gather_body(x_hbm, src_hbm, out_hbm, buf, idx, sem):
    c = lax.axis_index("c")
    s = lax.axis_index("s")
    wid = s * 2 + c
    base = wid * RPW
    pltpu.sync_copy(src_hbm.at[pl.ds(base, RPW)], idx)
    pltpu.async_copy(x_hbm.at[idx], buf, sem).wait()
    pltpu.sync_copy(buf, out_hbm.at[pl.ds(base, RPW)])


@functools.cache
def _make_gather():
    return functools.partial(
        pl.kernel,
        mesh=plsc.VectorSubcoreMesh(core_axis_name="c", subcore_axis_name="s"),
        out_type=jax.ShapeDtypeStruct((B * UNM, C), jnp.float32),
        scratch_types=[pltpu.VMEM((RPW, C), jnp.float32),
                       pltpu.VMEM((RPW,), jnp.int32),
                       pltpu.SemaphoreType.DMA],
    )(_gather_body)


def kernel(x):
    b_new, src = _prep(x.reshape(B, T1, 2, C))
    src_unm = src.reshape(B, T1)[:, RM:].reshape(B * UNM)
    unm = _make_gather()(x.reshape(2 * B * T1, C), src_unm)
    return jnp.concatenate([unm.reshape(B, UNM, C), b_new], axis=1)
